# baseline (speedup 1.0000x reference)
import functools

import jax
import jax.numpy as jnp
import numpy as np
from jax import lax
from jax.experimental import pallas as pl
from jax.experimental.pallas import tpu as pltpu
from jax.experimental.pallas import tpu_sc as plsc

VOCAB = 100000
HIDDEN = 64
BATCH = 4096
SEQ = 200
EPS = 1e-12

CHUNK = 128
NCHUNK = BATCH * SEQ // CHUNK
L = 16
GROUPS = CHUNK // L

def _rsqrt16(x):
    i = plsc.bitcast(x, jnp.int32)
    i = jnp.int32(0x5F3759DF) - (i >> 1)
    y = plsc.bitcast(i, jnp.float32)
    for _ in range(3):
        y = y * (1.5 - 0.5 * x * y * y)
    return y


def _make_kernel():
    info = plsc.get_sparse_core_info()
    nc, ns = info.num_cores, info.num_subcores
    nw = nc * ns
    chunks_per_w = NCHUNK // nw
    mesh = plsc.VectorSubcoreMesh(core_axis_name="c", subcore_axis_name="s")

    @functools.partial(
        pl.kernel,
        mesh=mesh,
        compiler_params=pltpu.CompilerParams(
            use_tc_tiling_on_sc=False, needs_layout_passes=False),
        out_type=jax.ShapeDtypeStruct((NCHUNK, CHUNK, HIDDEN), jnp.float32),
        scratch_types=[
            pltpu.VMEM((CHUNK,), jnp.int32),
            pltpu.VMEM((CHUNK, HIDDEN), jnp.float32),
            pltpu.VMEM((SEQ, HIDDEN), jnp.float32),
            pltpu.VMEM((HIDDEN, L), jnp.float32),
            pltpu.VMEM((HIDDEN,), jnp.float32),
            pltpu.VMEM((HIDDEN,), jnp.float32),
            pltpu.SemaphoreType.DMA,
        ],
    )
    def emb_ln(ids_hbm, tok_hbm, pos_hbm, gamma_hbm, beta_hbm, out_hbm,
               idx_v, buf_v, pos_v, xt_v, g_v, b_v, sem):
        wid = lax.axis_index("s") * nc + lax.axis_index("c")

        pltpu.sync_copy(pos_hbm.at[pl.ds(0, SEQ)], pos_v)
        pltpu.sync_copy(gamma_hbm, g_v)
        pltpu.sync_copy(beta_hbm, b_v)

        iota = lax.iota(jnp.int32, L)
        gvec = [g_v[pl.ds(q * L, L)] for q in range(HIDDEN // L)]
        bvec = [b_v[pl.ds(q * L, L)] for q in range(HIDDEN // L)]

        def do_chunk(ci, _):
            chunk = wid * chunks_per_w + ci
            pltpu.sync_copy(ids_hbm.at[chunk], idx_v)
            pltpu.async_copy(tok_hbm.at[idx_v], buf_v, sem).wait()

            def do_group(g, _):
                t0 = g * L
                rows = t0 + iota
                pos_idx = (chunk * CHUNK + rows) % SEQ
                acc_s = jnp.zeros((L,), jnp.float32)
                acc_q = jnp.zeros((L,), jnp.float32)
                for h in range(HIDDEN):
                    hv = jnp.full((L,), h, dtype=jnp.int32)
                    x = (plsc.load_gather(buf_v, [rows, hv])
                         + plsc.load_gather(pos_v, [pos_idx, hv]))
                    xt_v[h] = x
                    acc_s = acc_s + x
                    acc_q = acc_q + x * x
                mean = acc_s * (1.0 / HIDDEN)
                var = acc_q * (1.0 / HIDDEN) - mean * mean
                inv = _rsqrt16(var + EPS)
                for h in range(HIDDEN):
                    hv = jnp.full((L,), h, dtype=jnp.int32)
                    gs = gvec[h // L][h % L]
                    bs = bvec[h // L][h % L]
                    y = (xt_v[h] - mean) * inv * gs + bs
                    plsc.store_scatter(buf_v, [rows, hv], y)
                return ()

            lax.fori_loop(0, GROUPS, do_group, (), unroll=False)
            pltpu.sync_copy(buf_v, out_hbm.at[chunk])
            return ()

        lax.fori_loop(0, chunks_per_w, do_chunk, (), unroll=False)

    return emb_ln


_EMB_LN = _make_kernel()


@jax.jit
def kernel(input_ids, token_table, pos_table, gamma, beta):
    ids = input_ids.astype(jnp.int32).reshape(NCHUNK, CHUNK)
    out = _EMB_LN(ids, token_table, pos_table, gamma, beta)
    return out.reshape(BATCH, SEQ, HIDDEN)
